# Initial kernel scaffold; baseline (speedup 1.0000x reference)
import jax
import jax.numpy as jnp
from jax.experimental import pallas as pl


def kernel(regime_vector, pattern_prototypes, W1, b1, W2, b2):
    raise NotImplementedError("write your pallas kernel here")


# speedup vs baseline: 14.2434x; 14.2434x over previous
import functools

import jax
import jax.numpy as jnp
from jax.experimental import pallas as pl
from jax.experimental.pallas import tpu as pltpu

_B, _D, _N, _TOPK = 4096, 2048, 64, 3
_TB = 256


def _tile_kernel(x_ref, protos_ref, w1x_ref, w1p_ref, b1_ref, w2_ref, b2_ref,
                 out_ref):
    x = x_ref[...]
    protos = protos_ref[...]

    pscale = jax.lax.rsqrt(
        jnp.maximum(jnp.sum(protos * protos, axis=1, keepdims=True), 1e-24))
    pn = protos * pscale
    xscale = jax.lax.rsqrt(
        jnp.maximum(jnp.sum(x * x, axis=1, keepdims=True), 1e-24))

    sims = jax.lax.dot_general(x, pn, (((1,), (1,)), ((), ())),
                               preferred_element_type=jnp.float32)
    sims = sims * xscale

    iota = jax.lax.broadcasted_iota(jnp.int32, sims.shape, 1)
    cur = sims
    top_vals, top_idx = [], []
    for _ in range(_TOPK):
        m = jnp.max(cur, axis=1, keepdims=True)
        idx = jnp.min(jnp.where(cur >= m, iota, _N), axis=1,
                      keepdims=True)
        top_vals.append(m)
        top_idx.append(idx)
        cur = jnp.where(iota == idx, -jnp.inf, cur)

    es = [jnp.exp(5.0 * (v - top_vals[0])) for v in top_vals]
    denom = es[0] + es[1] + es[2]
    wts = [e / denom for e in es]

    ppw1 = jax.lax.dot_general(protos, w1p_ref[...], (((1,), (0,)), ((), ())),
                               preferred_element_type=jnp.float32)
    xw1 = jax.lax.dot_general(x, w1x_ref[...], (((1,), (0,)), ((), ())),
                              preferred_element_type=jnp.float32)
    b1 = b1_ref[...]

    hsum = jnp.zeros_like(xw1)
    for k in range(_TOPK):
        onehot = (iota == top_idx[k]).astype(jnp.float32)
        pk = jax.lax.dot_general(onehot, ppw1, (((1,), (0,)), ((), ())),
                                 preferred_element_type=jnp.float32)
        hsum = hsum + wts[k] * jnp.maximum(xw1 + pk + b1, 0.0)

    out = jax.lax.dot_general(hsum, w2_ref[...], (((1,), (0,)), ((), ())),
                              preferred_element_type=jnp.float32)
    out_ref[...] = out + b2_ref[...]


@jax.jit
def kernel(regime_vector, pattern_prototypes, W1, b1, W2, b2):
    w1x = W1[:_D]
    w1p = W1[_D:]
    b1r = b1.reshape(1, 32)
    b2r = b2.reshape(1, 16)
    grid = (_B // _TB,)
    return pl.pallas_call(
        _tile_kernel,
        grid=grid,
        in_specs=[
            pl.BlockSpec((_TB, _D), lambda i: (i, 0)),
            pl.BlockSpec((_N, _D), lambda i: (0, 0)),
            pl.BlockSpec((_D, 32), lambda i: (0, 0)),
            pl.BlockSpec((_D, 32), lambda i: (0, 0)),
            pl.BlockSpec((1, 32), lambda i: (0, 0)),
            pl.BlockSpec((32, 16), lambda i: (0, 0)),
            pl.BlockSpec((1, 16), lambda i: (0, 0)),
        ],
        out_specs=pl.BlockSpec((_TB, 16), lambda i: (i, 0)),
        out_shape=jax.ShapeDtypeStruct((_B, 16), jnp.float32),
        compiler_params=pltpu.CompilerParams(
            dimension_semantics=("arbitrary",)),
    )(regime_vector, pattern_prototypes, w1x, w1p, b1r, W2, b2r)
